# baseline (speedup 1.0000x reference)
import functools
import itertools

import jax
import jax.numpy as jnp
import numpy as np
from jax import lax
from jax.experimental import pallas as pl
from jax.experimental.pallas import tpu as pltpu
from jax.experimental.pallas import tpu_sc as plsc

L = 16
F = 2
N_MIN = 16
LOG2_T = 19
T = 2 ** LOG2_T
N_PTS = 524288
_B_SCALE = np.exp(np.log(512.0 / N_MIN) / (L - 1))
RES = [int(np.floor(N_MIN * (_B_SCALE ** l))) for l in range(L)]
DENSE = [(r + 1) ** 3 <= T for r in RES]
P1 = int(np.int32(np.uint32(2654435761)))
P2 = int(np.int32(np.uint32(805459861)))

NC = 2
NS = 16
LANES = 16
NW = NC * NS
PTS_PER_W = N_PTS // NW
CHUNK = 1024
NCHUNKS = PTS_PER_W // CHUNK
VPC = CHUNK // LANES


def _enc_body(xr, yr, zr, tab, h_out, xb, yb, zb, idxb0, idxb1, wb,
              rows0, rows1, outb, sem):
    wid = lax.axis_index("s") * NC + lax.axis_index("c")

    def chunk_body(ci, carry):
        base = wid * PTS_PER_W + ci * CHUNK
        pltpu.sync_copy(xr.at[pl.ds(base, CHUNK)], xb)
        pltpu.sync_copy(yr.at[pl.ds(base, CHUNK)], yb)
        pltpu.sync_copy(zr.at[pl.ds(base, CHUNK)], zb)

        for l in range(L):
            res = RES[l]
            dense = DENSE[l]
            s_half = 0.5 * float(res)

            def pass1(i, c1, l=l, res=res, dense=dense, s_half=s_half):
                s = i * LANES
                px = (xb[pl.ds(s, LANES)] + 1.0) * s_half
                py = (yb[pl.ds(s, LANES)] + 1.0) * s_half
                pz = (zb[pl.ds(s, LANES)] + 1.0) * s_half
                ix0 = px.astype(jnp.int32)
                iy0 = py.astype(jnp.int32)
                iz0 = pz.astype(jnp.int32)
                fx = px - ix0.astype(jnp.float32)
                fy = py - iy0.astype(jnp.float32)
                fz = pz - iz0.astype(jnp.float32)
                ix1 = jnp.minimum(ix0 + 1, res)
                iy1 = jnp.minimum(iy0 + 1, res)
                iz1 = jnp.minimum(iz0 + 1, res)
                wx = (1.0 - fx, fx)
                wy = (1.0 - fy, fy)
                wz = (1.0 - fz, fz)
                if dense:
                    S = res + 1
                    bx = (ix0, ix1)
                    by = (iy0 * S, iy1 * S)
                    bz = (iz0 * (S * S) + l * T, iz1 * (S * S) + l * T)
                else:
                    bx = (ix0, ix1)
                    by = (iy0 * P1, iy1 * P1)
                    bz = (iz0 * P2, iz1 * P2)
                wxy = {(cx, cy): wx[cx] * wy[cy] for cx in (0, 1) for cy in (0, 1)}
                for c, (cx, cy, cz) in enumerate(
                        itertools.product((0, 1), (0, 1), (0, 1))):
                    if dense:
                        idx = (bx[cx] + by[cy]) + bz[cz]
                    else:
                        idx = ((bx[cx] ^ by[cy] ^ bz[cz]) & (T - 1)) + l * T
                    w = wxy[(cx, cy)] * wz[cz]
                    e0 = idx * 2
                    idxb0[pl.ds(i * 128 + 16 * c, 16)] = e0
                    idxb1[pl.ds(i * 128 + 16 * c, 16)] = e0 + 1
                    wb[pl.ds(i * 128 + 16 * c, 16)] = w
                return c1

            lax.fori_loop(0, VPC, pass1, 0)

            cp0 = pltpu.async_copy(tab.at[idxb0], rows0, sem)
            cp1 = pltpu.async_copy(tab.at[idxb1], rows1, sem)
            cp0.wait()
            cp1.wait()

            def pass2(i, c2, l=l):
                acc0 = jnp.zeros((16,), jnp.float32)
                acc1 = jnp.zeros((16,), jnp.float32)
                for c in range(8):
                    q = i * 128 + 16 * c
                    r0 = rows0[pl.ds(q, 16)]
                    r1 = rows1[pl.ds(q, 16)]
                    w = wb[pl.ds(q, 16)]
                    acc0 = acc0 + w * r0
                    acc1 = acc1 + w * r1
                outb[2 * l, pl.ds(i * LANES, LANES)] = acc0
                outb[2 * l + 1, pl.ds(i * LANES, LANES)] = acc1
                return c2

            lax.fori_loop(0, VPC, pass2, 0)

        pltpu.sync_copy(outb, h_out.at[:, pl.ds(base, CHUNK)])
        return carry

    lax.fori_loop(0, NCHUNKS, chunk_body, 0)


def _make_encoder():
    mesh = plsc.VectorSubcoreMesh(core_axis_name="c", subcore_axis_name="s")
    return functools.partial(
        pl.kernel,
        mesh=mesh,
        out_type=jax.ShapeDtypeStruct((2 * L, N_PTS), jnp.float32),
        scratch_types=[
            pltpu.VMEM((CHUNK,), jnp.float32),
            pltpu.VMEM((CHUNK,), jnp.float32),
            pltpu.VMEM((CHUNK,), jnp.float32),
            pltpu.VMEM((CHUNK * 8,), jnp.int32),
            pltpu.VMEM((CHUNK * 8,), jnp.int32),
            pltpu.VMEM((CHUNK * 8,), jnp.float32),
            pltpu.VMEM((CHUNK * 8,), jnp.float32),
            pltpu.VMEM((CHUNK * 8,), jnp.float32),
            pltpu.VMEM((2 * L, CHUNK), jnp.float32),
            pltpu.SemaphoreType.DMA,
        ],
    )(_enc_body)


_FREQ18 = (np.pi * (2.0 ** (np.arange(18) // 3))).astype(np.float32).reshape(18, 1)

MLP_B = 2048


def _mlp_body(h_ref, n_ref, f18_ref, v1h_ref, v1s_ref, v1c_ref, g1_ref, b1_ref,
              v2_ref, g2_ref, b2_ref, o_ref):
    nb = n_ref[...]
    ang = jnp.broadcast_to(nb[None], (6, 3, MLP_B)).reshape(18, MLP_B)
    ang = ang * f18_ref[...]
    s18 = jnp.sin(ang)
    c18 = jnp.cos(ang)
    v1h = v1h_ref[...]
    v1s = v1s_ref[...]
    v1c = v1c_ref[...]
    nrm2 = (jnp.sum(v1h * v1h, axis=1, keepdims=True)
            + jnp.sum(v1s * v1s, axis=1, keepdims=True)
            + jnp.sum(v1c * v1c, axis=1, keepdims=True))
    scale1 = g1_ref[...] * jax.lax.rsqrt(nrm2)
    acc = jax.lax.dot_general(v1h, h_ref[...], (((1,), (0,)), ((), ())),
                              preferred_element_type=jnp.float32)
    acc = acc + jax.lax.dot_general(v1s, s18, (((1,), (0,)), ((), ())),
                                    preferred_element_type=jnp.float32)
    acc = acc + jax.lax.dot_general(v1c, c18, (((1,), (0,)), ((), ())),
                                    preferred_element_type=jnp.float32)
    hid = jnp.maximum(acc * scale1 + b1_ref[...], 0.0)
    v2 = v2_ref[...]
    nrm2b = jnp.sum(v2 * v2, axis=1, keepdims=True)
    scale2 = g2_ref[...] * jax.lax.rsqrt(nrm2b)
    out = jax.lax.dot_general(v2, hid, (((1,), (0,)), ((), ())),
                              preferred_element_type=jnp.float32)
    o_ref[...] = jnp.tanh(out * scale2 + b2_ref[...])


def _mlp(h_t, n_t, v1h, v1s, v1c, g1c, b1c, v2p, g2p, b2p):
    grid = (N_PTS // MLP_B,)
    full = lambda shape: pl.BlockSpec(shape, lambda i: (0, 0))
    return pl.pallas_call(
        _mlp_body,
        grid=grid,
        in_specs=[
            pl.BlockSpec((2 * L, MLP_B), lambda i: (0, i)),
            pl.BlockSpec((3, MLP_B), lambda i: (0, i)),
            full((18, 1)),
            full((64, 2 * L)),
            full((64, 18)),
            full((64, 18)),
            full((64, 1)),
            full((64, 1)),
            full((8, 64)),
            full((8, 1)),
            full((8, 1)),
        ],
        out_specs=pl.BlockSpec((8, MLP_B), lambda i: (0, i)),
        out_shape=jax.ShapeDtypeStruct((8, N_PTS), jnp.float32),
    )(h_t, n_t, jnp.asarray(_FREQ18), v1h, v1s, v1c, g1c, b1c, v2p, g2p, b2p)


def kernel(x, n, table, V1, g1, b1, V2, g2, b2):
    nt = n.T
    tab = table.reshape(L * T * F)
    h_t = _make_encoder()(x[:, 0], x[:, 1], x[:, 2], tab)

    sin_cols = [2 * L + d * 12 + f * 2 + 0 for f in range(6) for d in range(3)]
    cos_cols = [2 * L + d * 12 + f * 2 + 1 for f in range(6) for d in range(3)]
    v1h = V1[:, : 2 * L]
    v1s = V1[:, np.array(sin_cols)]
    v1c = V1[:, np.array(cos_cols)]
    g1c = g1[:, None]
    b1c = b1[:, None]
    v2p = jnp.concatenate([V2, jnp.ones((5, 64), jnp.float32)], axis=0)
    g2p = jnp.concatenate([g2, jnp.zeros((5,), jnp.float32)])[:, None]
    b2p = jnp.concatenate([b2, jnp.zeros((5,), jnp.float32)])[:, None]

    o_t = _mlp(h_t, nt, v1h, v1s, v1c, g1c, b1c, v2p, g2p, b2p)
    return o_t[:3].T


# speedup vs baseline: 2.5831x; 2.5831x over previous
import functools
import itertools

import jax
import jax.numpy as jnp
import numpy as np
from jax import lax
from jax.experimental import pallas as pl
from jax.experimental.pallas import tpu as pltpu
from jax.experimental.pallas import tpu_sc as plsc

L = 16
F = 2
N_MIN = 16
LOG2_T = 19
T = 2 ** LOG2_T
N_PTS = 524288
_B_SCALE = np.exp(np.log(512.0 / N_MIN) / (L - 1))
RES = [int(np.floor(N_MIN * (_B_SCALE ** l))) for l in range(L)]
DENSE = [(r + 1) ** 3 <= T for r in RES]
P1 = int(np.int32(np.uint32(2654435761)))
P2 = int(np.int32(np.uint32(805459861)))

NC = 2
NS = 16
LANES = 16
NW = NC * NS
PTS_PER_W = N_PTS // NW
CHUNK = 1024
NCHUNKS = PTS_PER_W // CHUNK
VPC = CHUNK // LANES


def _enc_body(xr, yr, zr, tab, h_out, xb, yb, zb, idxb0, idxb1, wb,
              rows0, rows1, outb, sem):
    wid = lax.axis_index("s") * NC + lax.axis_index("c")

    def chunk_body(ci, carry):
        base = wid * PTS_PER_W + ci * CHUNK
        pltpu.sync_copy(xr.at[pl.ds(base, CHUNK)], xb)
        pltpu.sync_copy(yr.at[pl.ds(base, CHUNK)], yb)
        pltpu.sync_copy(zr.at[pl.ds(base, CHUNK)], zb)

        for l in range(L):
            res = RES[l]
            dense = DENSE[l]
            s_half = 0.5 * float(res)

            def pass1(i, c1, l=l, res=res, dense=dense, s_half=s_half):
                s = i * LANES
                px = (xb[pl.ds(s, LANES)] + 1.0) * s_half
                py = (yb[pl.ds(s, LANES)] + 1.0) * s_half
                pz = (zb[pl.ds(s, LANES)] + 1.0) * s_half
                ix0 = px.astype(jnp.int32)
                iy0 = py.astype(jnp.int32)
                iz0 = pz.astype(jnp.int32)
                fx = px - ix0.astype(jnp.float32)
                fy = py - iy0.astype(jnp.float32)
                fz = pz - iz0.astype(jnp.float32)
                ix1 = jnp.minimum(ix0 + 1, res)
                iy1 = jnp.minimum(iy0 + 1, res)
                iz1 = jnp.minimum(iz0 + 1, res)
                wx = (1.0 - fx, fx)
                wy = (1.0 - fy, fy)
                wz = (1.0 - fz, fz)
                if dense:
                    S = res + 1
                    bx = (ix0, ix1)
                    by = (iy0 * S, iy1 * S)
                    bz = (iz0 * (S * S), iz1 * (S * S))
                else:
                    bx = (ix0, ix1)
                    by = (iy0 * P1, iy1 * P1)
                    bz = (iz0 * P2, iz1 * P2)
                wxy = {(cx, cy): wx[cx] * wy[cy] for cx in (0, 1) for cy in (0, 1)}
                for c, (cx, cy, cz) in enumerate(
                        itertools.product((0, 1), (0, 1), (0, 1))):
                    if dense:
                        t = (bx[cx] + by[cy]) + bz[cz]
                    else:
                        t = (bx[cx] ^ by[cy] ^ bz[cz]) & (T - 1)
                    w = wxy[(cx, cy)] * wz[cz]
                    e0 = ((t >> 7) << 8) + (t & 127) + (l * 2 * T)
                    idxb0[pl.ds(i * 128 + 16 * c, 16)] = e0
                    idxb1[pl.ds(i * 128 + 16 * c, 16)] = e0 + 128
                    wb[pl.ds(i * 128 + 16 * c, 16)] = w
                return c1

            lax.fori_loop(0, VPC, pass1, 0)

            cp0 = pltpu.async_copy(tab.at[idxb0], rows0, sem)
            cp1 = pltpu.async_copy(tab.at[idxb1], rows1, sem)
            cp0.wait()
            cp1.wait()

            def pass2(i, c2, l=l):
                acc0 = jnp.zeros((16,), jnp.float32)
                acc1 = jnp.zeros((16,), jnp.float32)
                for c in range(8):
                    q = i * 128 + 16 * c
                    r0 = rows0[pl.ds(q, 16)]
                    r1 = rows1[pl.ds(q, 16)]
                    w = wb[pl.ds(q, 16)]
                    acc0 = acc0 + w * r0
                    acc1 = acc1 + w * r1
                outb[2 * l, pl.ds(i * LANES, LANES)] = acc0
                outb[2 * l + 1, pl.ds(i * LANES, LANES)] = acc1
                return c2

            lax.fori_loop(0, VPC, pass2, 0)

        pltpu.sync_copy(outb, h_out.at[:, pl.ds(base, CHUNK)])
        return carry

    lax.fori_loop(0, NCHUNKS, chunk_body, 0)


def _make_encoder():
    mesh = plsc.VectorSubcoreMesh(core_axis_name="c", subcore_axis_name="s")
    return functools.partial(
        pl.kernel,
        mesh=mesh,
        out_type=jax.ShapeDtypeStruct((2 * L, N_PTS), jnp.float32),
        scratch_types=[
            pltpu.VMEM((CHUNK,), jnp.float32),
            pltpu.VMEM((CHUNK,), jnp.float32),
            pltpu.VMEM((CHUNK,), jnp.float32),
            pltpu.VMEM((CHUNK * 8,), jnp.int32),
            pltpu.VMEM((CHUNK * 8,), jnp.int32),
            pltpu.VMEM((CHUNK * 8,), jnp.float32),
            pltpu.VMEM((CHUNK * 8,), jnp.float32),
            pltpu.VMEM((CHUNK * 8,), jnp.float32),
            pltpu.VMEM((2 * L, CHUNK), jnp.float32),
            pltpu.SemaphoreType.DMA,
        ],
    )(_enc_body)


_FREQ18 = (np.pi * (2.0 ** (np.arange(18) // 3))).astype(np.float32).reshape(18, 1)

MLP_B = 2048


def _mlp_body(h_ref, n_ref, f18_ref, v1h_ref, v1s_ref, v1c_ref, g1_ref, b1_ref,
              v2_ref, g2_ref, b2_ref, o_ref):
    nb = n_ref[...]
    ang = jnp.broadcast_to(nb[None], (6, 3, MLP_B)).reshape(18, MLP_B)
    ang = ang * f18_ref[...]
    s18 = jnp.sin(ang)
    c18 = jnp.cos(ang)
    v1h = v1h_ref[...]
    v1s = v1s_ref[...]
    v1c = v1c_ref[...]
    nrm2 = (jnp.sum(v1h * v1h, axis=1, keepdims=True)
            + jnp.sum(v1s * v1s, axis=1, keepdims=True)
            + jnp.sum(v1c * v1c, axis=1, keepdims=True))
    scale1 = g1_ref[...] * jax.lax.rsqrt(nrm2)
    acc = jax.lax.dot_general(v1h, h_ref[...], (((1,), (0,)), ((), ())),
                              preferred_element_type=jnp.float32)
    acc = acc + jax.lax.dot_general(v1s, s18, (((1,), (0,)), ((), ())),
                                    preferred_element_type=jnp.float32)
    acc = acc + jax.lax.dot_general(v1c, c18, (((1,), (0,)), ((), ())),
                                    preferred_element_type=jnp.float32)
    hid = jnp.maximum(acc * scale1 + b1_ref[...], 0.0)
    v2 = v2_ref[...]
    nrm2b = jnp.sum(v2 * v2, axis=1, keepdims=True)
    scale2 = g2_ref[...] * jax.lax.rsqrt(nrm2b)
    out = jax.lax.dot_general(v2, hid, (((1,), (0,)), ((), ())),
                              preferred_element_type=jnp.float32)
    o_ref[...] = jnp.tanh(out * scale2 + b2_ref[...])


def _mlp(h_t, n_t, v1h, v1s, v1c, g1c, b1c, v2p, g2p, b2p):
    grid = (N_PTS // MLP_B,)
    full = lambda shape: pl.BlockSpec(shape, lambda i: (0, 0))
    return pl.pallas_call(
        _mlp_body,
        grid=grid,
        in_specs=[
            pl.BlockSpec((2 * L, MLP_B), lambda i: (0, i)),
            pl.BlockSpec((3, MLP_B), lambda i: (0, i)),
            full((18, 1)),
            full((64, 2 * L)),
            full((64, 18)),
            full((64, 18)),
            full((64, 1)),
            full((64, 1)),
            full((8, 64)),
            full((8, 1)),
            full((8, 1)),
        ],
        out_specs=pl.BlockSpec((8, MLP_B), lambda i: (0, i)),
        out_shape=jax.ShapeDtypeStruct((8, N_PTS), jnp.float32),
    )(h_t, n_t, jnp.asarray(_FREQ18), v1h, v1s, v1c, g1c, b1c, v2p, g2p, b2p)


def kernel(x, n, table, V1, g1, b1, V2, g2, b2):
    nt = n.T
    tab = table.reshape(L, T // 128, 128, F).transpose(0, 1, 3, 2).reshape(-1)
    h_t = _make_encoder()(x[:, 0], x[:, 1], x[:, 2], tab)

    sin_cols = [2 * L + d * 12 + f * 2 + 0 for f in range(6) for d in range(3)]
    cos_cols = [2 * L + d * 12 + f * 2 + 1 for f in range(6) for d in range(3)]
    v1h = V1[:, : 2 * L]
    v1s = V1[:, np.array(sin_cols)]
    v1c = V1[:, np.array(cos_cols)]
    g1c = g1[:, None]
    b1c = b1[:, None]
    v2p = jnp.concatenate([V2, jnp.ones((5, 64), jnp.float32)], axis=0)
    g2p = jnp.concatenate([g2, jnp.zeros((5,), jnp.float32)])[:, None]
    b2p = jnp.concatenate([b2, jnp.zeros((5,), jnp.float32)])[:, None]

    o_t = _mlp(h_t, nt, v1h, v1s, v1c, g1c, b1c, v2p, g2p, b2p)
    return o_t[:3].T


# speedup vs baseline: 2.8875x; 1.1179x over previous
import functools
import itertools

import jax
import jax.numpy as jnp
import numpy as np
from jax import lax
from jax.experimental import pallas as pl
from jax.experimental.pallas import tpu as pltpu
from jax.experimental.pallas import tpu_sc as plsc

L = 16
F = 2
N_MIN = 16
LOG2_T = 19
T = 2 ** LOG2_T
N_PTS = 524288
_B_SCALE = np.exp(np.log(512.0 / N_MIN) / (L - 1))
RES = [int(np.floor(N_MIN * (_B_SCALE ** l))) for l in range(L)]
DENSE = [(r + 1) ** 3 <= T for r in RES]
P1 = int(np.int32(np.uint32(2654435761)))
P2 = int(np.int32(np.uint32(805459861)))

NC = 2
NS = 16
LANES = 16
NW = NC * NS
PTS_PER_W = N_PTS // NW
CHUNK = 1024
NCHUNKS = PTS_PER_W // CHUNK
VPC = CHUNK // LANES


def _enc_body(xr, yr, zr, tab, h_out, xb, yb, zb,
              idx0a, idx0b, idx1a, idx1b, wba, wbb,
              r0a, r0b, r1a, r1b, outb, sema, semb):
    wid = lax.axis_index("s") * NC + lax.axis_index("c")
    idx0 = (idx0a, idx0b)
    idx1 = (idx1a, idx1b)
    wbs = (wba, wbb)
    rows0 = (r0a, r0b)
    rows1 = (r1a, r1b)
    sems = (sema, semb)

    def chunk_body(ci, carry):
        base = wid * PTS_PER_W + ci * CHUNK
        pltpu.sync_copy(xr.at[pl.ds(base, CHUNK)], xb)
        pltpu.sync_copy(yr.at[pl.ds(base, CHUNK)], yb)
        pltpu.sync_copy(zr.at[pl.ds(base, CHUNK)], zb)

        def run_pass1(l, p):
            res = RES[l]
            dense = DENSE[l]
            s_half = 0.5 * float(res)
            ib0, ib1, wb = idx0[p], idx1[p], wbs[p]

            def pass1(i, c1):
                s = i * LANES
                px = (xb[pl.ds(s, LANES)] + 1.0) * s_half
                py = (yb[pl.ds(s, LANES)] + 1.0) * s_half
                pz = (zb[pl.ds(s, LANES)] + 1.0) * s_half
                ix0 = px.astype(jnp.int32)
                iy0 = py.astype(jnp.int32)
                iz0 = pz.astype(jnp.int32)
                fx = px - ix0.astype(jnp.float32)
                fy = py - iy0.astype(jnp.float32)
                fz = pz - iz0.astype(jnp.float32)
                ix1 = jnp.minimum(ix0 + 1, res)
                iy1 = jnp.minimum(iy0 + 1, res)
                iz1 = jnp.minimum(iz0 + 1, res)
                wx = (1.0 - fx, fx)
                wy = (1.0 - fy, fy)
                wz = (1.0 - fz, fz)
                if dense:
                    S = res + 1
                    bx = (ix0, ix1)
                    by = (iy0 * S, iy1 * S)
                    bz = (iz0 * (S * S), iz1 * (S * S))
                else:
                    bx = (ix0, ix1)
                    by = (iy0 * P1, iy1 * P1)
                    bz = (iz0 * P2, iz1 * P2)
                wxy = {(cx, cy): wx[cx] * wy[cy] for cx in (0, 1) for cy in (0, 1)}
                for c, (cx, cy, cz) in enumerate(
                        itertools.product((0, 1), (0, 1), (0, 1))):
                    if dense:
                        t = (bx[cx] + by[cy]) + bz[cz]
                    else:
                        t = (bx[cx] ^ by[cy] ^ bz[cz]) & (T - 1)
                    w = wxy[(cx, cy)] * wz[cz]
                    e0 = ((t >> 7) << 8) + (t & 127) + (l * 2 * T)
                    ib0[pl.ds(i * 128 + 16 * c, 16)] = e0
                    ib1[pl.ds(i * 128 + 16 * c, 16)] = e0 + 128
                    wb[pl.ds(i * 128 + 16 * c, 16)] = w
                return c1

            lax.fori_loop(0, VPC, pass1, 0)

        def run_pass2(l, p):
            wb, rb0, rb1 = wbs[p], rows0[p], rows1[p]

            def pass2(i, c2):
                acc0 = jnp.zeros((16,), jnp.float32)
                acc1 = jnp.zeros((16,), jnp.float32)
                for c in range(8):
                    q = i * 128 + 16 * c
                    r0 = rb0[pl.ds(q, 16)]
                    r1 = rb1[pl.ds(q, 16)]
                    w = wb[pl.ds(q, 16)]
                    acc0 = acc0 + w * r0
                    acc1 = acc1 + w * r1
                outb[2 * l, pl.ds(i * LANES, LANES)] = acc0
                outb[2 * l + 1, pl.ds(i * LANES, LANES)] = acc1
                return c2

            lax.fori_loop(0, VPC, pass2, 0)

        pend = [None, None]
        for l in range(L + 1):
            if l < L:
                p = l & 1
                run_pass1(l, p)
                cp0 = pltpu.async_copy(tab.at[idx0[p]], rows0[p], sems[p])
                cp1 = pltpu.async_copy(tab.at[idx1[p]], rows1[p], sems[p])
                pend[p] = (cp0, cp1)
            if l >= 1:
                q = (l - 1) & 1
                c0, c1 = pend[q]
                c0.wait()
                c1.wait()
                run_pass2(l - 1, q)

        pltpu.sync_copy(outb, h_out.at[:, pl.ds(base, CHUNK)])
        return carry

    lax.fori_loop(0, NCHUNKS, chunk_body, 0)


def _make_encoder():
    mesh = plsc.VectorSubcoreMesh(core_axis_name="c", subcore_axis_name="s")
    return functools.partial(
        pl.kernel,
        mesh=mesh,
        out_type=jax.ShapeDtypeStruct((2 * L, N_PTS), jnp.float32),
        scratch_types=[
            pltpu.VMEM((CHUNK,), jnp.float32),
            pltpu.VMEM((CHUNK,), jnp.float32),
            pltpu.VMEM((CHUNK,), jnp.float32),
            pltpu.VMEM((CHUNK * 8,), jnp.int32),
            pltpu.VMEM((CHUNK * 8,), jnp.int32),
            pltpu.VMEM((CHUNK * 8,), jnp.int32),
            pltpu.VMEM((CHUNK * 8,), jnp.int32),
            pltpu.VMEM((CHUNK * 8,), jnp.float32),
            pltpu.VMEM((CHUNK * 8,), jnp.float32),
            pltpu.VMEM((CHUNK * 8,), jnp.float32),
            pltpu.VMEM((CHUNK * 8,), jnp.float32),
            pltpu.VMEM((CHUNK * 8,), jnp.float32),
            pltpu.VMEM((CHUNK * 8,), jnp.float32),
            pltpu.VMEM((2 * L, CHUNK), jnp.float32),
            pltpu.SemaphoreType.DMA,
            pltpu.SemaphoreType.DMA,
        ],
    )(_enc_body)


_FREQ18 = (np.pi * (2.0 ** (np.arange(18) // 3))).astype(np.float32).reshape(18, 1)

MLP_B = 2048


def _mlp_body(h_ref, n_ref, f18_ref, v1h_ref, v1s_ref, v1c_ref, g1_ref, b1_ref,
              v2_ref, g2_ref, b2_ref, o_ref):
    nb = n_ref[...]
    ang = jnp.broadcast_to(nb[None], (6, 3, MLP_B)).reshape(18, MLP_B)
    ang = ang * f18_ref[...]
    s18 = jnp.sin(ang)
    c18 = jnp.cos(ang)
    v1h = v1h_ref[...]
    v1s = v1s_ref[...]
    v1c = v1c_ref[...]
    nrm2 = (jnp.sum(v1h * v1h, axis=1, keepdims=True)
            + jnp.sum(v1s * v1s, axis=1, keepdims=True)
            + jnp.sum(v1c * v1c, axis=1, keepdims=True))
    scale1 = g1_ref[...] * jax.lax.rsqrt(nrm2)
    acc = jax.lax.dot_general(v1h, h_ref[...], (((1,), (0,)), ((), ())),
                              preferred_element_type=jnp.float32)
    acc = acc + jax.lax.dot_general(v1s, s18, (((1,), (0,)), ((), ())),
                                    preferred_element_type=jnp.float32)
    acc = acc + jax.lax.dot_general(v1c, c18, (((1,), (0,)), ((), ())),
                                    preferred_element_type=jnp.float32)
    hid = jnp.maximum(acc * scale1 + b1_ref[...], 0.0)
    v2 = v2_ref[...]
    nrm2b = jnp.sum(v2 * v2, axis=1, keepdims=True)
    scale2 = g2_ref[...] * jax.lax.rsqrt(nrm2b)
    out = jax.lax.dot_general(v2, hid, (((1,), (0,)), ((), ())),
                              preferred_element_type=jnp.float32)
    o_ref[...] = jnp.tanh(out * scale2 + b2_ref[...])


def _mlp(h_t, n_t, v1h, v1s, v1c, g1c, b1c, v2p, g2p, b2p):
    grid = (N_PTS // MLP_B,)
    full = lambda shape: pl.BlockSpec(shape, lambda i: (0, 0))
    return pl.pallas_call(
        _mlp_body,
        grid=grid,
        in_specs=[
            pl.BlockSpec((2 * L, MLP_B), lambda i: (0, i)),
            pl.BlockSpec((3, MLP_B), lambda i: (0, i)),
            full((18, 1)),
            full((64, 2 * L)),
            full((64, 18)),
            full((64, 18)),
            full((64, 1)),
            full((64, 1)),
            full((8, 64)),
            full((8, 1)),
            full((8, 1)),
        ],
        out_specs=pl.BlockSpec((8, MLP_B), lambda i: (0, i)),
        out_shape=jax.ShapeDtypeStruct((8, N_PTS), jnp.float32),
    )(h_t, n_t, jnp.asarray(_FREQ18), v1h, v1s, v1c, g1c, b1c, v2p, g2p, b2p)


def kernel(x, n, table, V1, g1, b1, V2, g2, b2):
    nt = n.T
    tab = table.reshape(L, T // 128, 128, F).transpose(0, 1, 3, 2).reshape(-1)
    h_t = _make_encoder()(x[:, 0], x[:, 1], x[:, 2], tab)

    sin_cols = [2 * L + d * 12 + f * 2 + 0 for f in range(6) for d in range(3)]
    cos_cols = [2 * L + d * 12 + f * 2 + 1 for f in range(6) for d in range(3)]
    v1h = V1[:, : 2 * L]
    v1s = V1[:, np.array(sin_cols)]
    v1c = V1[:, np.array(cos_cols)]
    g1c = g1[:, None]
    b1c = b1[:, None]
    v2p = jnp.concatenate([V2, jnp.ones((5, 64), jnp.float32)], axis=0)
    g2p = jnp.concatenate([g2, jnp.zeros((5,), jnp.float32)])[:, None]
    b2p = jnp.concatenate([b2, jnp.zeros((5,), jnp.float32)])[:, None]

    o_t = _mlp(h_t, nt, v1h, v1s, v1c, g1c, b1c, v2p, g2p, b2p)
    return o_t[:3].T
